# baseline (speedup 1.0000x reference)
import jax
import jax.numpy as jnp
from jax.experimental import pallas as pl

_TB = 128


def _rvq_body(xt_ref, et_ref, e_ref, agg_ref, ind_ref):
    tb, d = xt_ref.shape
    c_num, _, k = et_ref.shape
    x_res = xt_ref[:]
    z_q = jnp.zeros_like(x_res)
    iota_k = jax.lax.broadcasted_iota(jnp.int32, (tb, k), 1)
    for c in range(c_num):
        e_t = et_ref[c]
        e2 = jnp.sum(e_t * e_t, axis=0, keepdims=True)
        x2 = jnp.sum(x_res * x_res, axis=1, keepdims=True)
        p = jnp.dot(x_res, e_t)
        d2 = (x2 - 2.0 * p) + e2
        m = jnp.min(d2, axis=1, keepdims=True)
        ind = jnp.min(jnp.where(d2 == m, iota_k, k), axis=1, keepdims=True)
        oh = (iota_k == ind).astype(jnp.float32)
        sel = jax.lax.dot_general(
            oh, e_ref[c], (((1,), (0,)), ((), ())),
            precision=jax.lax.Precision.HIGHEST)
        x_res = x_res - sel
        z_q = z_q + sel
        agg_ref[c] = z_q
        ind_ref[c] = ind[:, 0]


def kernel(x_in, code_embeddings):
    b, d, t = x_in.shape
    c_num, k, _ = code_embeddings.shape
    nt = b * t
    xt = jnp.transpose(x_in, (0, 2, 1)).reshape(nt, d)
    e_t = jnp.transpose(code_embeddings, (0, 2, 1))
    grid = (pl.cdiv(nt, _TB),)
    aggs, inds = pl.pallas_call(
        _rvq_body,
        grid=grid,
        in_specs=[
            pl.BlockSpec((_TB, d), lambda i: (i, 0)),
            pl.BlockSpec((c_num, d, k), lambda i: (0, 0, 0)),
            pl.BlockSpec((c_num, k, d), lambda i: (0, 0, 0)),
        ],
        out_specs=[
            pl.BlockSpec((c_num, _TB, d), lambda i: (0, i, 0)),
            pl.BlockSpec((c_num, _TB), lambda i: (0, i)),
        ],
        out_shape=[
            jax.ShapeDtypeStruct((c_num, nt, d), jnp.float32),
            jax.ShapeDtypeStruct((c_num, nt), jnp.int32),
        ],
    )(xt, e_t, code_embeddings)
    z_q_aggregated = jnp.transpose(aggs.reshape(c_num, b, t, d), (1, 0, 3, 2))
    indices = jnp.transpose(inds.reshape(c_num, b, t), (1, 2, 0))
    return z_q_aggregated, indices


# speedup vs baseline: 1.9253x; 1.9253x over previous
import jax
import jax.numpy as jnp
from jax.experimental import pallas as pl
from jax.experimental.pallas import tpu as pltpu

_TB = 256


def _rvq_body(xt_ref, et_ref, es_ref, agg_ref, ind_ref, e2_ref):
    tb, d = xt_ref.shape
    c_num, _, k = et_ref.shape
    h = tb // 2

    @pl.when(pl.program_id(0) == 0)
    def _():
        for c in range(c_num):
            e_t = et_ref[c]
            e2_ref[c, :] = jnp.sum(e_t * e_t, axis=0)

    iota_f = jax.lax.broadcasted_iota(jnp.int32, (h, k), 1).astype(jnp.float32)
    xs = [xt_ref[:h], xt_ref[h:]]
    zqs = [jnp.zeros((h, d), jnp.float32) for _ in range(2)]
    for c in range(c_num):
        e_t = et_ref[c]
        e2 = e2_ref[c:c + 1, :]
        es = es_ref[c]
        for j in range(2):
            x_res = xs[j]
            x2 = jnp.sum(x_res * x_res, axis=1, keepdims=True)
            p = jnp.dot(x_res, e_t)
            d2 = (x2 - 2.0 * p) + e2
            m = jnp.min(d2, axis=1, keepdims=True)
            indf = jnp.min(jnp.where(d2 == m, iota_f, float(k)),
                           axis=1, keepdims=True)
            oh = (iota_f == indf).astype(jnp.bfloat16)
            parts = jax.lax.dot_general(
                oh, es, (((1,), (0,)), ((), ())),
                preferred_element_type=jnp.float32)
            sel = ((parts[:, :d] + parts[:, d:2 * d])
                   + parts[:, 2 * d:3 * d]) + parts[:, 3 * d:]
            xs[j] = x_res - sel
            zqs[j] = zqs[j] + sel
            agg_ref[c, j * h:(j + 1) * h] = zqs[j]
            ind_ref[c, j * h:(j + 1) * h] = indf[:, 0].astype(jnp.int32)


def kernel(x_in, code_embeddings):
    b, d, t = x_in.shape
    c_num, k, _ = code_embeddings.shape
    nt = b * t
    xt = jnp.transpose(x_in, (0, 2, 1)).reshape(nt, d)
    e_t = jnp.transpose(code_embeddings, (0, 2, 1))
    hi = code_embeddings.astype(jnp.bfloat16)
    r1 = code_embeddings - hi.astype(jnp.float32)
    mid = r1.astype(jnp.bfloat16)
    r2 = r1 - mid.astype(jnp.float32)
    lo = r2.astype(jnp.bfloat16)
    r3 = r2 - lo.astype(jnp.float32)
    lo2 = r3.astype(jnp.bfloat16)
    e_split = jnp.concatenate([hi, mid, lo, lo2], axis=-1)
    grid = (pl.cdiv(nt, _TB),)
    aggs, inds = pl.pallas_call(
        _rvq_body,
        grid=grid,
        in_specs=[
            pl.BlockSpec((_TB, d), lambda i: (i, 0)),
            pl.BlockSpec((c_num, d, k), lambda i: (0, 0, 0)),
            pl.BlockSpec((c_num, k, 4 * d), lambda i: (0, 0, 0)),
        ],
        out_specs=[
            pl.BlockSpec((c_num, _TB, d), lambda i: (0, i, 0)),
            pl.BlockSpec((c_num, _TB), lambda i: (0, i)),
        ],
        out_shape=[
            jax.ShapeDtypeStruct((c_num, nt, d), jnp.float32),
            jax.ShapeDtypeStruct((c_num, nt), jnp.int32),
        ],
        scratch_shapes=[pltpu.VMEM((c_num, k), jnp.float32)],
    )(xt, e_t, e_split)
    z_q_aggregated = jnp.transpose(aggs.reshape(c_num, b, t, d), (1, 0, 3, 2))
    indices = jnp.transpose(inds.reshape(c_num, b, t), (1, 2, 0))
    return z_q_aggregated, indices
